# baseline (speedup 1.0000x reference)
import functools

import jax
import jax.numpy as jnp
from jax import lax
from jax.experimental import pallas as pl
from jax.experimental.pallas import tpu as pltpu
from jax.experimental.pallas import tpu_sc as plsc

N = 10000
E = 320000
D = 128
NC = 2
NS = 16
NW = NC * NS
L = 16

EPT = E // NW
K = 128
NCHUNK = 78
TAIL = EPT - NCHUNK * K
RPT = 624
SC0 = 24

_GDN = lax.GatherDimensionNumbers(
    offset_dims=(), collapsed_slice_dims=(0,), start_index_map=(0,))


def _bcast_lane(vec, i):
    idx = jnp.full((L, 1), i, jnp.int32)
    return lax.gather(vec, idx, dimension_numbers=_GDN, slice_sizes=(1,),
                      mode=lax.GatherScatterMode.PROMISE_IN_BOUNDS)


def _sc_body(h, src1, dst3, dstT3, ew1, p0, p1, acc, src_v, dst_v, ew_v,
             rows_v, srcT_v, dstT_v, ewT_v, sg, ss):
    c = lax.axis_index("c")
    s = lax.axis_index("s")
    w = c * NS + s

    zvec = jnp.zeros((L,), jnp.float32)

    def z_body(i, carry):
        for k in range(D // L):
            rows_v[0, i, pl.ds(k * L, L)] = zvec
        return carry

    lax.fori_loop(0, K, z_body, 0, unroll=4)
    rbase = s * RPT
    rem = N - NS * RPT
    zb = rows_v.at[0]
    for j in range(RPT // K):
        pltpu.sync_copy(zb, acc.at[pl.ds(rbase + j * K, K)])
    pltpu.sync_copy(zb.at[pl.ds(0, RPT - (RPT // K) * K)],
                    acc.at[pl.ds(rbase + (RPT // K) * K,
                                 RPT - (RPT // K) * K)])

    @pl.when(s == NS - 1)
    def _():
        pltpu.sync_copy(zb.at[pl.ds(0, rem)],
                        acc.at[pl.ds(NS * RPT, rem)])

    plsc.subcore_barrier()


    def _scale(t, m):
        def grp_body(g, carry2):
            ewv = ew_v[pl.ds(t * K + g * L, L)]
            r0 = g * L
            for i in range(L):
                wv = _bcast_lane(ewv, i)
                for k in range(D // L):
                    sl = pl.ds(k * L, L)
                    rows_v[m, r0 + i, sl] = rows_v[m, r0 + i, sl] * wv
            return carry2

        lax.fori_loop(0, K // L, grp_body, 0)

    def pair_body(u, carry):
        a = 2 * u
        b = a + 1
        da = pltpu.async_copy(h.at[src_v.at[pl.ds(a * K, K)]],
                              rows_v.at[0], sg)
        db = pltpu.async_copy(h.at[src_v.at[pl.ds(b * K, K)]],
                              rows_v.at[1], sg)
        da.wait()
        _scale(a, 0)
        sa = pltpu.async_copy(rows_v.at[0], acc.at[dst_v.at[a]], ss,
                              add=True)
        db.wait()
        _scale(b, 1)
        sb = pltpu.async_copy(rows_v.at[1], acc.at[dst_v.at[b]], ss,
                              add=True)
        sa.wait()
        sb.wait()
        return carry

    for stage in range(4):
        n_ch = SC0 if stage < 3 else NCHUNK - 3 * SC0
        ebase = w * EPT + stage * SC0 * K
        pltpu.sync_copy(src1.at[pl.ds(ebase, n_ch * K)],
                        src_v.at[pl.ds(0, n_ch * K)])
        pltpu.sync_copy(dst3.at[w, pl.ds(stage * SC0, n_ch)],
                        dst_v.at[pl.ds(0, n_ch)])
        pltpu.sync_copy(ew1.at[pl.ds(ebase, n_ch * K)],
                        ew_v.at[pl.ds(0, n_ch * K)])

        lax.fori_loop(0, n_ch // 2, pair_body, 0)

    tbase = w * EPT + NCHUNK * K
    pltpu.sync_copy(src1.at[pl.ds(tbase, TAIL)], srcT_v)
    pltpu.sync_copy(dstT3.at[w], dstT_v)
    pltpu.sync_copy(ew1.at[pl.ds(tbase, TAIL)], ewT_v)
    pltpu.async_copy(h.at[srcT_v], rows_v.at[0, pl.ds(0, TAIL)], sg).wait()
    ewvT = ewT_v[...]
    for i in range(TAIL):
        wvT = _bcast_lane(ewvT, i)
        for k in range(D // L):
            sl = pl.ds(k * L, L)
            rows_v[0, i, sl] = rows_v[0, i, sl] * wvT
    pltpu.sync_copy(rows_v.at[0, pl.ds(0, TAIL)], acc.at[dstT_v.at[0]],
                    add=True)
    plsc.subcore_barrier()

    @pl.when(c == 0)
    def _():
        pltpu.sync_copy(acc.at[pl.ds(rbase, RPT)], p0.at[pl.ds(rbase, RPT)])

        @pl.when(s == NS - 1)
        def _():
            pltpu.sync_copy(acc.at[pl.ds(NS * RPT, rem)],
                            p0.at[pl.ds(NS * RPT, rem)])

    @pl.when(c == 1)
    def _():
        pltpu.sync_copy(acc.at[pl.ds(rbase, RPT)], p1.at[pl.ds(rbase, RPT)])

        @pl.when(s == NS - 1)
        def _():
            pltpu.sync_copy(acc.at[pl.ds(NS * RPT, rem)],
                            p1.at[pl.ds(NS * RPT, rem)])


_sc_aggregate = functools.partial(
    pl.kernel,
    out_type=(jax.ShapeDtypeStruct((N, D), jnp.float32),
              jax.ShapeDtypeStruct((N, D), jnp.float32)),
    mesh=plsc.VectorSubcoreMesh(core_axis_name="c", subcore_axis_name="s"),
    scratch_types=[
        pltpu.VMEM_SHARED((N, D), jnp.float32),
        pltpu.VMEM((SC0 * K,), jnp.int32),
        pltpu.VMEM((SC0, K), jnp.int32),
        pltpu.VMEM((SC0 * K,), jnp.float32),
        pltpu.VMEM((2, K, D), jnp.float32),
        pltpu.VMEM((TAIL,), jnp.int32),
        pltpu.VMEM((1, TAIL), jnp.int32),
        pltpu.VMEM((TAIL,), jnp.float32),
        pltpu.SemaphoreType.DMA,
        pltpu.SemaphoreType.DMA,
    ],
)(_sc_body)


def _mm_body(p0_ref, p1_ref, w_ref, o_ref):
    agg = p0_ref[...] + p1_ref[...]
    acc = jnp.dot(agg, w_ref[...], preferred_element_type=jnp.float32)
    o_ref[...] = jnp.maximum(acc, 0.0)


def _matmul_relu(p0, p1, weight):
    grid = 10
    rb = N // grid
    return pl.pallas_call(
        _mm_body,
        grid=(grid,),
        in_specs=[
            pl.BlockSpec((rb, D), lambda i: (i, 0)),
            pl.BlockSpec((rb, D), lambda i: (i, 0)),
            pl.BlockSpec((D, D), lambda i: (0, 0)),
        ],
        out_specs=pl.BlockSpec((rb, D), lambda i: (i, 0)),
        out_shape=jax.ShapeDtypeStruct((N, D), jnp.float32),
    )(p0, p1, weight)


@jax.jit
def kernel(h, edge_index, edge_weight, weight):
    src1 = edge_index[0].astype(jnp.int32)
    dst2 = edge_index[1].astype(jnp.int32).reshape(NW, EPT)
    dst3 = dst2[:, :NCHUNK * K].reshape(NW, NCHUNK, K)
    dstT3 = dst2[:, NCHUNK * K:].reshape(NW, 1, TAIL)
    p0, p1 = _sc_aggregate(h, src1, dst3, dstT3, edge_weight)
    return _matmul_relu(p0, p1, weight)


# speedup vs baseline: 1.0264x; 1.0264x over previous
import functools

import jax
import jax.numpy as jnp
from jax import lax
from jax.experimental import pallas as pl
from jax.experimental.pallas import tpu as pltpu
from jax.experimental.pallas import tpu_sc as plsc

N = 10000
E = 320000
D = 128
NC = 2
NS = 16
NW = NC * NS
L = 16

EPT = E // NW
K = 80
NCHUNK = EPT // K
RPT = 624
SC0 = 32

_GDN = lax.GatherDimensionNumbers(
    offset_dims=(), collapsed_slice_dims=(0,), start_index_map=(0,))


def _bcast_lane(vec, i):
    idx = jnp.full((L, 1), i, jnp.int32)
    return lax.gather(vec, idx, dimension_numbers=_GDN, slice_sizes=(1,),
                      mode=lax.GatherScatterMode.PROMISE_IN_BOUNDS)


def _sc_body(h, src1, dst3, ew1, p0, p1, acc, src_v, dst_v, ew_v, rows_v,
             sg, ss):
    c = lax.axis_index("c")
    s = lax.axis_index("s")
    w = c * NS + s

    zvec = jnp.zeros((L,), jnp.float32)

    def z_body(i, carry):
        for k in range(D // L):
            rows_v[0, i, pl.ds(k * L, L)] = zvec
        return carry

    lax.fori_loop(0, K, z_body, 0, unroll=4)
    rbase = s * RPT
    rem = N - NS * RPT
    zb = rows_v.at[0]
    for j in range(RPT // K):
        pltpu.sync_copy(zb, acc.at[pl.ds(rbase + j * K, K)])
    pltpu.sync_copy(zb.at[pl.ds(0, RPT - (RPT // K) * K)],
                    acc.at[pl.ds(rbase + (RPT // K) * K,
                                 RPT - (RPT // K) * K)])

    @pl.when(s == NS - 1)
    def _():
        pltpu.sync_copy(zb.at[pl.ds(0, rem)],
                        acc.at[pl.ds(NS * RPT, rem)])

    plsc.subcore_barrier()


    def _scale(t, m):
        def grp_body(g, carry2):
            ewv = ew_v[pl.ds(t * K + g * L, L)]
            r0 = g * L
            for i in range(L):
                wv = _bcast_lane(ewv, i)
                for k in range(D // L):
                    sl = pl.ds(k * L, L)
                    rows_v[m, r0 + i, sl] = rows_v[m, r0 + i, sl] * wv
            return carry2

        lax.fori_loop(0, K // L, grp_body, 0)

    def trip_body(u, carry):
        a = 3 * u
        ds_g = [pltpu.async_copy(h.at[src_v.at[pl.ds((a + j) * K, K)]],
                                 rows_v.at[j], sg) for j in range(3)]
        ds_s = []
        for j in range(3):
            ds_g[j].wait()
            _scale(a + j, j)
            ds_s.append(pltpu.async_copy(rows_v.at[j],
                                         acc.at[dst_v.at[a + j]], ss,
                                         add=True))
        for d in ds_s:
            d.wait()
        return carry

    for stage in range(4):
        n_ch = SC0 if stage < 3 else NCHUNK - 3 * SC0
        ebase = w * EPT + stage * SC0 * K
        pltpu.sync_copy(src1.at[pl.ds(ebase, n_ch * K)],
                        src_v.at[pl.ds(0, n_ch * K)])
        pltpu.sync_copy(dst3.at[w, pl.ds(stage * SC0, n_ch)],
                        dst_v.at[pl.ds(0, n_ch)])
        pltpu.sync_copy(ew1.at[pl.ds(ebase, n_ch * K)],
                        ew_v.at[pl.ds(0, n_ch * K)])

        lax.fori_loop(0, n_ch // 3, trip_body, 0)

        for tl in range(3 * (n_ch // 3), n_ch):
            pltpu.async_copy(h.at[src_v.at[pl.ds(tl * K, K)]],
                             rows_v.at[0], sg).wait()
            _scale(tl, 0)
            pltpu.sync_copy(rows_v.at[0], acc.at[dst_v.at[tl]], add=True)
    plsc.subcore_barrier()

    @pl.when(c == 0)
    def _():
        pltpu.sync_copy(acc.at[pl.ds(rbase, RPT)], p0.at[pl.ds(rbase, RPT)])

        @pl.when(s == NS - 1)
        def _():
            pltpu.sync_copy(acc.at[pl.ds(NS * RPT, rem)],
                            p0.at[pl.ds(NS * RPT, rem)])

    @pl.when(c == 1)
    def _():
        pltpu.sync_copy(acc.at[pl.ds(rbase, RPT)], p1.at[pl.ds(rbase, RPT)])

        @pl.when(s == NS - 1)
        def _():
            pltpu.sync_copy(acc.at[pl.ds(NS * RPT, rem)],
                            p1.at[pl.ds(NS * RPT, rem)])


_sc_aggregate = functools.partial(
    pl.kernel,
    out_type=(jax.ShapeDtypeStruct((N, D), jnp.float32),
              jax.ShapeDtypeStruct((N, D), jnp.float32)),
    mesh=plsc.VectorSubcoreMesh(core_axis_name="c", subcore_axis_name="s"),
    compiler_params=pltpu.CompilerParams(use_tc_tiling_on_sc=False),
    scratch_types=[
        pltpu.VMEM_SHARED((N, D), jnp.float32),
        pltpu.VMEM((SC0 * K,), jnp.int32),
        pltpu.VMEM((SC0, K), jnp.int32),
        pltpu.VMEM((SC0 * K,), jnp.float32),
        pltpu.VMEM((3, K, D), jnp.float32),
        pltpu.SemaphoreType.DMA,
        pltpu.SemaphoreType.DMA,
    ],
)(_sc_body)


def _mm_body(p0_ref, p1_ref, w_ref, o_ref):
    agg = p0_ref[...] + p1_ref[...]
    acc = jnp.dot(agg, w_ref[...], preferred_element_type=jnp.float32)
    o_ref[...] = jnp.maximum(acc, 0.0)


def _matmul_relu(p0, p1, weight):
    grid = 10
    rb = N // grid
    return pl.pallas_call(
        _mm_body,
        grid=(grid,),
        in_specs=[
            pl.BlockSpec((rb, D), lambda i: (i, 0)),
            pl.BlockSpec((rb, D), lambda i: (i, 0)),
            pl.BlockSpec((D, D), lambda i: (0, 0)),
        ],
        out_specs=pl.BlockSpec((rb, D), lambda i: (i, 0)),
        out_shape=jax.ShapeDtypeStruct((N, D), jnp.float32),
    )(p0, p1, weight)


@jax.jit
def kernel(h, edge_index, edge_weight, weight):
    src1 = edge_index[0].astype(jnp.int32)
    dst3 = edge_index[1].astype(jnp.int32).reshape(NW, NCHUNK, K)
    p0, p1 = _sc_aggregate(h, src1, dst3, edge_weight)
    return _matmul_relu(p0, p1, weight)
